# baseline (speedup 1.0000x reference)
import functools

import jax
import jax.numpy as jnp
from jax.experimental import pallas as pl

_MIN_KEPT = 100000
_THRESH_BITS = 0x3F333333
_NLOG_THRESH = 0.35667494393873245
_RH = 256


def _loss_body(x_ref, t_ref, acc_ref):
    x = x_ref[0]
    t = t_ref[0]
    m = jnp.max(x, axis=0)
    s = jnp.sum(jnp.exp(x - m), axis=0)
    lse = m + jnp.log(s)
    cls = jax.lax.broadcasted_iota(jnp.int32, x.shape, 0)
    logit_t = jnp.sum(jnp.where(cls == t[None], x, 0.0), axis=0)
    loss = lse - logit_t

    thr = jnp.float32(_NLOG_THRESH)
    lt = loss > thr
    le = loss >= thr

    @pl.when((pl.program_id(0) == 0) & (pl.program_id(1) == 0))
    def _():
        acc_ref[...] = jnp.zeros_like(acc_ref)

    acc_ref[0] += jnp.where(lt, loss, 0.0)
    acc_ref[1] += lt.astype(jnp.float32)
    acc_ref[2] += le.astype(jnp.float32)
    return loss


def _acc_kernel(x_ref, t_ref, acc_ref):
    _loss_body(x_ref, t_ref, acc_ref)


def _loss_kernel(x_ref, t_ref, loss_ref, acc_ref):
    loss_ref[0] = _loss_body(x_ref, t_ref, acc_ref)


def _bisect_kernel(loss_ref, out_ref, *, kept):
    loss = loss_ref[...]
    pbits = jax.lax.bitcast_convert_type(jnp.exp(-loss), jnp.int32)

    def body(_, carry):
        lo, hi = carry
        mid = jax.lax.div(lo + hi, jnp.int32(2))
        cnt = jnp.sum((pbits <= mid).astype(jnp.int32))
        ge = cnt >= jnp.int32(kept + 1)
        return jnp.where(ge, lo, mid + 1), jnp.where(ge, mid, hi)

    lo, _ = jax.lax.fori_loop(0, 31, body,
                              (jnp.int32(0), jnp.int32(0x40000000)))
    thr_bits = jnp.maximum(lo, jnp.int32(_THRESH_BITS))
    sel = pbits < thr_bits
    total = jnp.sum(jnp.where(sel, loss, 0.0))
    count = jnp.sum(sel.astype(jnp.int32))
    res = total / jnp.maximum(count, 1).astype(jnp.float32)
    out_ref[...] = jnp.broadcast_to(res, (1, 1))


def kernel(predict, target):
    B, ncls, H, W = predict.shape
    nbh = H // _RH
    kept = _MIN_KEPT * B

    grid = (B, nbh)
    in_specs = [
        pl.BlockSpec((1, ncls, _RH, W), lambda b, j: (b, 0, j, 0)),
        pl.BlockSpec((1, _RH, W), lambda b, j: (b, j, 0)),
    ]
    acc_spec = pl.BlockSpec((3, _RH, W), lambda b, j: (0, 0, 0))
    acc_shape = jax.ShapeDtypeStruct((3, _RH, W), jnp.float32)

    acc = pl.pallas_call(
        _acc_kernel,
        grid=grid,
        in_specs=in_specs,
        out_specs=acc_spec,
        out_shape=acc_shape,
    )(predict, target)

    s_lt = jnp.sum(acc[0])
    c_lt = jnp.sum(acc[1])
    c_le = jnp.sum(acc[2])

    def fast_path(*_):
        return s_lt / jnp.maximum(c_lt, 1.0)

    def bisect_path(predict_, target_):
        rh = _RH // 2
        loss3, _ = pl.pallas_call(
            _loss_kernel,
            grid=(B, H // rh),
            in_specs=[
                pl.BlockSpec((1, ncls, rh, W), lambda b, j: (b, 0, j, 0)),
                pl.BlockSpec((1, rh, W), lambda b, j: (b, j, 0)),
            ],
            out_specs=[
                pl.BlockSpec((1, rh, W), lambda b, j: (b, j, 0)),
                pl.BlockSpec((3, rh, W), lambda b, j: (0, 0, 0)),
            ],
            out_shape=[
                jax.ShapeDtypeStruct((B, H, W), jnp.float32),
                jax.ShapeDtypeStruct((3, rh, W), jnp.float32),
            ],
        )(predict_, target_)
        out = pl.pallas_call(
            functools.partial(_bisect_kernel, kept=kept),
            out_shape=jax.ShapeDtypeStruct((1, 1), jnp.float32),
        )(loss3.reshape(B * H, W))
        return out[0, 0]

    return jax.lax.cond(c_le < jnp.float32(kept + 1), bisect_path, fast_path,
                        predict, target)


# speedup vs baseline: 1.3043x; 1.3043x over previous
import functools

import jax
import jax.numpy as jnp
from jax.experimental import pallas as pl

_MIN_KEPT = 100000
_THRESH_BITS = 0x3F333333
_NLOG_THRESH = 0.35667494393873245
_RH = 512


def _loss_kernel(x_ref, t_ref, loss_ref, acc_ref):
    x = x_ref[0]
    t = t_ref[0]
    m = jnp.max(x, axis=0)
    s = jnp.sum(jnp.exp(x - m), axis=0)
    lse = m + jnp.log(s)
    cls = jax.lax.broadcasted_iota(jnp.int32, x.shape, 0)
    logit_t = jnp.sum(jnp.where(cls == t[None], x, 0.0), axis=0)
    loss = lse - logit_t
    loss_ref[0] = loss

    thr = jnp.float32(_NLOG_THRESH)
    lt = loss > thr
    le = loss >= thr

    @pl.when((pl.program_id(0) == 0) & (pl.program_id(1) == 0))
    def _():
        acc_ref[...] = jnp.zeros_like(acc_ref)

    acc_ref[0] += jnp.where(lt, loss, 0.0)
    acc_ref[1] += lt.astype(jnp.float32)
    acc_ref[2] += le.astype(jnp.float32)


def _bisect_kernel(loss_ref, out_ref, *, kept):
    loss = loss_ref[...]
    pbits = jax.lax.bitcast_convert_type(jnp.exp(-loss), jnp.int32)

    def body(_, carry):
        lo, hi = carry
        mid = jax.lax.div(lo + hi, jnp.int32(2))
        cnt = jnp.sum((pbits <= mid).astype(jnp.int32))
        ge = cnt >= jnp.int32(kept + 1)
        return jnp.where(ge, lo, mid + 1), jnp.where(ge, mid, hi)

    lo, _ = jax.lax.fori_loop(0, 31, body,
                              (jnp.int32(0), jnp.int32(0x40000000)))
    thr_bits = jnp.maximum(lo, jnp.int32(_THRESH_BITS))
    sel = pbits < thr_bits
    total = jnp.sum(jnp.where(sel, loss, 0.0))
    count = jnp.sum(sel.astype(jnp.int32))
    res = total / jnp.maximum(count, 1).astype(jnp.float32)
    out_ref[...] = jnp.broadcast_to(res, (1, 1))


def kernel(predict, target):
    B, ncls, H, W = predict.shape
    nbh = H // _RH
    kept = _MIN_KEPT * B

    loss3, acc = pl.pallas_call(
        _loss_kernel,
        grid=(B, nbh),
        in_specs=[
            pl.BlockSpec((1, ncls, _RH, W), lambda b, j: (b, 0, j, 0)),
            pl.BlockSpec((1, _RH, W), lambda b, j: (b, j, 0)),
        ],
        out_specs=[
            pl.BlockSpec((1, _RH, W), lambda b, j: (b, j, 0)),
            pl.BlockSpec((3, _RH, W), lambda b, j: (0, 0, 0)),
        ],
        out_shape=[
            jax.ShapeDtypeStruct((B, H, W), jnp.float32),
            jax.ShapeDtypeStruct((3, _RH, W), jnp.float32),
        ],
    )(predict, target)

    s_lt = jnp.sum(acc[0])
    c_lt = jnp.sum(acc[1])
    c_le = jnp.sum(acc[2])

    def fast_path(_):
        return s_lt / jnp.maximum(c_lt, 1.0)

    def bisect_path(loss3_):
        out = pl.pallas_call(
            functools.partial(_bisect_kernel, kept=kept),
            out_shape=jax.ShapeDtypeStruct((1, 1), jnp.float32),
        )(loss3_.reshape(B * H, W))
        return out[0, 0]

    return jax.lax.cond(c_le < jnp.float32(kept + 1), bisect_path, fast_path,
                        loss3)
